# baseline (speedup 1.0000x reference)
import jax
import jax.numpy as jnp
from jax import lax
from jax.experimental import pallas as pl
from jax.experimental.pallas import tpu as pltpu
from jax.experimental.pallas import tpu_sc as plsc

D_MODEL = 1024
NC = 2
NS = 16
NW = NC * NS

K = 16
NBUF = 7


def _gather_body(x_hbm, pe_hbm, out_hbm, idx_v, *bufs_and_sems):
    bufs = bufs_and_sems[:NBUF]
    gsems = bufs_and_sems[NBUF:2 * NBUF]
    wsems = bufs_and_sems[2 * NBUF:3 * NBUF]
    c = lax.axis_index("c")
    s = lax.axis_index("s")
    wid = s * NC + c
    n_chunks = idx_v.shape[0]
    n_per_w = n_chunks * idx_v.shape[1]

    def out_slice(j):
        return out_hbm.at[pl.ds(wid * n_per_w + j * K, K)]

    def wait_gather_and_writeback(j):
        q = j % NBUF
        pltpu.make_async_copy(pe_hbm.at[idx_v.at[j]], bufs[q], gsems[q]).wait()
        pltpu.async_copy(bufs[q], out_slice(j), wsems[q])

    pltpu.sync_copy(x_hbm.at[wid], idx_v)
    for j in range(n_chunks):
        p = j % NBUF
        if j >= NBUF:
            pltpu.make_async_copy(bufs[p], out_slice(j - NBUF), wsems[p]).wait()
        pltpu.async_copy(pe_hbm.at[idx_v.at[j]], bufs[p], gsems[p])
        if j >= 1:
            wait_gather_and_writeback(j - 1)
    wait_gather_and_writeback(n_chunks - 1)
    for j in range(max(0, n_chunks - NBUF), n_chunks):
        p = j % NBUF
        pltpu.make_async_copy(bufs[p], out_slice(j), wsems[p]).wait()


def kernel(x, pe):
    b, l = x.shape
    total = b * l
    n_per_w = total // NW
    n_chunks = n_per_w // K
    x_resh = x.reshape(NW, n_chunks, K).astype(jnp.int32)

    mesh = plsc.VectorSubcoreMesh(core_axis_name="c", subcore_axis_name="s")
    out = pl.kernel(
        _gather_body,
        out_type=jax.ShapeDtypeStruct((total, D_MODEL), jnp.float32),
        mesh=mesh,
        scratch_types=(
            [pltpu.VMEM((n_chunks, K), jnp.int32)]
            + [pltpu.VMEM((K, D_MODEL), jnp.float32) for _ in range(NBUF)]
            + [pltpu.SemaphoreType.DMA for _ in range(2 * NBUF)]
        ),
    )(x_resh, pe)
    return out.reshape(b, l, D_MODEL)


# speedup vs baseline: 1.0166x; 1.0166x over previous
import jax
import jax.numpy as jnp
from jax import lax
from jax.experimental import pallas as pl
from jax.experimental.pallas import tpu as pltpu
from jax.experimental.pallas import tpu_sc as plsc

D_MODEL = 1024
NC = 2
NS = 16
NW = NC * NS

K = 32
NBUF = 3


def _gather_body(x_hbm, pe_hbm, out_hbm, idx_v, *bufs_and_sems):
    bufs = bufs_and_sems[:NBUF]
    gsems = bufs_and_sems[NBUF:2 * NBUF]
    wsems = bufs_and_sems[2 * NBUF:3 * NBUF]
    c = lax.axis_index("c")
    s = lax.axis_index("s")
    wid = s * NC + c
    n_chunks = idx_v.shape[0]
    n_per_w = n_chunks * idx_v.shape[1]

    def out_slice(j):
        return out_hbm.at[pl.ds(wid * n_per_w + j * K, K)]

    def wait_gather_and_writeback(j):
        q = j % NBUF
        pltpu.make_async_copy(pe_hbm.at[idx_v.at[j]], bufs[q], gsems[q]).wait()
        pltpu.async_copy(bufs[q], out_slice(j), wsems[q])

    pltpu.sync_copy(x_hbm.at[wid], idx_v)
    for j in range(n_chunks):
        p = j % NBUF
        if j >= NBUF:
            pltpu.make_async_copy(bufs[p], out_slice(j - NBUF), wsems[p]).wait()
        pltpu.async_copy(pe_hbm.at[idx_v.at[j]], bufs[p], gsems[p])
        if j >= 2:
            wait_gather_and_writeback(j - 2)
    for j in range(max(0, n_chunks - 2), n_chunks):
        wait_gather_and_writeback(j)
    for j in range(max(0, n_chunks - NBUF), n_chunks):
        p = j % NBUF
        pltpu.make_async_copy(bufs[p], out_slice(j), wsems[p]).wait()


def kernel(x, pe):
    b, l = x.shape
    total = b * l
    n_per_w = total // NW
    n_chunks = n_per_w // K
    x_resh = x.reshape(NW, n_chunks, K).astype(jnp.int32)

    mesh = plsc.VectorSubcoreMesh(core_axis_name="c", subcore_axis_name="s")
    out = pl.kernel(
        _gather_body,
        out_type=jax.ShapeDtypeStruct((total, D_MODEL), jnp.float32),
        mesh=mesh,
        scratch_types=(
            [pltpu.VMEM((n_chunks, K), jnp.int32)]
            + [pltpu.VMEM((K, D_MODEL), jnp.float32) for _ in range(NBUF)]
            + [pltpu.SemaphoreType.DMA for _ in range(2 * NBUF)]
        ),
    )(x_resh, pe)
    return out.reshape(b, l, D_MODEL)
